# initial kernel scaffold (unmeasured)
import jax
import jax.numpy as jnp
from jax import lax
from jax.experimental import pallas as pl
from jax.experimental.pallas import tpu as pltpu

P = 8
M_BLK = 512
K_BLK = 512


def kernel(x, w_mat):
    m_glob, k_shard = x.shape
    _k_glob, n = w_mat.shape

    def body(x_ref, w_ref, out_ref, xbf_ref, comm_ref, send_sems, recv_sems):
        my = lax.axis_index("i")

        barrier = pltpu.get_barrier_semaphore()
        for t in range(1, P):
            pl.semaphore_signal(
                barrier, inc=1,
                device_id=(lax.rem(my + t, P),),
                device_id_type=pl.DeviceIdType.MESH,
            )
        pl.semaphore_wait(barrier, P - 1)

        xbf_ref[...] = x_ref[...].astype(jnp.bfloat16)

        sends = []
        for t in range(1, P):
            dst = lax.rem(my + t, P)
            rdma = pltpu.make_async_remote_copy(
                src_ref=xbf_ref.at[pl.ds(dst * M_BLK, M_BLK), :],
                dst_ref=comm_ref.at[t - 1],
                send_sem=send_sems.at[t - 1],
                recv_sem=recv_sems.at[t - 1],
                device_id=(dst,),
                device_id_type=pl.DeviceIdType.MESH,
            )
            rdma.start()
            sends.append(rdma)

        own = xbf_ref[pl.ds(my * M_BLK, M_BLK), :]
        wblk = w_ref[pl.ds(my * K_BLK, K_BLK), :].astype(jnp.bfloat16)
        out_ref[...] = jnp.dot(own, wblk, preferred_element_type=jnp.float32)

        for t in range(1, P):
            recv = pltpu.make_async_remote_copy(
                src_ref=comm_ref.at[t - 1],
                dst_ref=comm_ref.at[t - 1],
                send_sem=send_sems.at[t - 1],
                recv_sem=recv_sems.at[t - 1],
                device_id=(my,),
                device_id_type=pl.DeviceIdType.MESH,
            )
            recv.wait_recv()
            src = lax.rem(my - t + P, P)
            wblk = w_ref[pl.ds(src * K_BLK, K_BLK), :].astype(jnp.bfloat16)
            out_ref[...] += jnp.dot(
                comm_ref[t - 1], wblk, preferred_element_type=jnp.float32
            )

        for rdma in sends:
            rdma.wait_send()

        y = out_ref[...]
        c = 0.7978845608028654
        out_ref[...] = 0.5 * y * (1.0 + jnp.tanh(c * (y + 0.044715 * y * y * y)))

    return pl.pallas_call(
        body,
        out_shape=jax.ShapeDtypeStruct((m_glob // P, n), jnp.float32),
        in_specs=[
            pl.BlockSpec(memory_space=pltpu.VMEM),
            pl.BlockSpec(memory_space=pltpu.VMEM),
        ],
        out_specs=pl.BlockSpec(memory_space=pltpu.VMEM),
        scratch_shapes=[
            pltpu.VMEM((m_glob, k_shard), jnp.bfloat16),
            pltpu.VMEM((P - 1, M_BLK, k_shard), jnp.bfloat16),
            pltpu.SemaphoreType.DMA((P - 1,)),
            pltpu.SemaphoreType.DMA((P - 1,)),
        ],
        compiler_params=pltpu.CompilerParams(collective_id=0),
    )(x, w_mat)


# baseline (device time: 60471 ns/iter reference)
import jax
import jax.numpy as jnp
from jax import lax
from jax.experimental import pallas as pl
from jax.experimental.pallas import tpu as pltpu

P = 8
M_BLK = 512
K_BLK = 512


def kernel(x, w_mat):
    m_glob, k_shard = x.shape
    _k_glob, n = w_mat.shape

    def body(x_ref, w_ref, out_ref, xbf_ref, comm_ref, send_sems, recv_sems):
        my = lax.axis_index("i")

        barrier = pltpu.get_barrier_semaphore()
        for t in range(1, P):
            pl.semaphore_signal(
                barrier, inc=1,
                device_id=(lax.rem(my + t, P),),
                device_id_type=pl.DeviceIdType.MESH,
            )
        pl.semaphore_wait(barrier, P - 1)

        xbf_ref[...] = x_ref[...].astype(jnp.bfloat16)

        sends = []
        for t in range(1, P):
            dst = lax.rem(my + t, P)
            rdma = pltpu.make_async_remote_copy(
                src_ref=xbf_ref.at[pl.ds(dst * M_BLK, M_BLK), :],
                dst_ref=comm_ref.at[t - 1],
                send_sem=send_sems.at[t - 1],
                recv_sem=recv_sems.at[t - 1],
                device_id=(dst,),
                device_id_type=pl.DeviceIdType.MESH,
            )
            rdma.start()
            sends.append(rdma)

        own = xbf_ref[pl.ds(my * M_BLK, M_BLK), :]
        wblk = w_ref[pl.ds(my * K_BLK, K_BLK), :].astype(jnp.bfloat16)
        out_ref[...] = jnp.dot(own, wblk, preferred_element_type=jnp.float32)

        for t in range(1, P):
            recv = pltpu.make_async_remote_copy(
                src_ref=comm_ref.at[t - 1],
                dst_ref=comm_ref.at[t - 1],
                send_sem=send_sems.at[t - 1],
                recv_sem=recv_sems.at[t - 1],
                device_id=(my,),
                device_id_type=pl.DeviceIdType.MESH,
            )
            recv.wait_recv()
            src = lax.rem(my - t + P, P)
            wblk = w_ref[pl.ds(src * K_BLK, K_BLK), :].astype(jnp.bfloat16)
            out_ref[...] += jnp.dot(
                comm_ref[t - 1], wblk, preferred_element_type=jnp.float32
            )

        for rdma in sends:
            rdma.wait_send()

        y = out_ref[...]
        c = 0.7978845608028654
        out_ref[...] = 0.5 * y * (1.0 + jnp.tanh(c * (y + 0.044715 * y * y * y)))

    return pl.pallas_call(
        body,
        out_shape=jax.ShapeDtypeStruct((m_glob // P, n), jnp.float32),
        in_specs=[
            pl.BlockSpec(memory_space=pltpu.VMEM),
            pl.BlockSpec(memory_space=pltpu.VMEM),
        ],
        out_specs=pl.BlockSpec(memory_space=pltpu.VMEM),
        scratch_shapes=[
            pltpu.VMEM((m_glob, k_shard), jnp.bfloat16),
            pltpu.VMEM((P - 1, M_BLK, k_shard), jnp.bfloat16),
            pltpu.SemaphoreType.DMA((P - 1,)),
            pltpu.SemaphoreType.DMA((P - 1,)),
        ],
        compiler_params=pltpu.CompilerParams(
            collective_id=0,
            vmem_limit_bytes=100 * 1024 * 1024,
        ),
    )(x, w_mat)


# device time: 50010 ns/iter; 1.2092x vs baseline; 1.2092x over previous
import jax
import jax.numpy as jnp
from jax import lax
from jax.experimental import pallas as pl
from jax.experimental.pallas import tpu as pltpu

P = 8
M_BLK = 512
K_BLK = 512
NBUF = 3


def kernel(x, w_mat):
    m_glob, k_shard = x.shape
    _k_glob, n = w_mat.shape

    def body(x_hbm, w_hbm, out_ref, xf_ref, xbf_ref, wbuf_ref, comm_ref,
             xsem, wsems, send_sems, recv_sems):
        my = lax.axis_index("i")

        xcopy = pltpu.make_async_copy(x_hbm, xf_ref, xsem)
        xcopy.start()

        def w_copy(idx):
            s = lax.rem(my - idx + P, P)
            return pltpu.make_async_copy(
                w_hbm.at[pl.ds(s * K_BLK, K_BLK), :],
                wbuf_ref.at[idx % NBUF],
                wsems.at[idx % NBUF],
            )

        wcopies = {}
        for idx in range(NBUF):
            wcopies[idx] = w_copy(idx)
            wcopies[idx].start()

        barrier = pltpu.get_barrier_semaphore()
        for t in range(1, P):
            pl.semaphore_signal(
                barrier, inc=1,
                device_id=(lax.rem(my + t, P),),
                device_id_type=pl.DeviceIdType.MESH,
            )
        pl.semaphore_wait(barrier, P - 1)

        xcopy.wait()
        xbf_ref[...] = xf_ref[...].astype(jnp.bfloat16)

        sends = []
        for t in range(1, P):
            dst = lax.rem(my + t, P)
            rdma = pltpu.make_async_remote_copy(
                src_ref=xbf_ref.at[pl.ds(dst * M_BLK, M_BLK), :],
                dst_ref=comm_ref.at[t - 1],
                send_sem=send_sems.at[t - 1],
                recv_sem=recv_sems.at[t - 1],
                device_id=(dst,),
                device_id_type=pl.DeviceIdType.MESH,
            )
            rdma.start()
            sends.append(rdma)

        for t in range(P):
            wcopies[t].wait()
            if t == 0:
                a = xbf_ref[pl.ds(my * M_BLK, M_BLK), :]
            else:
                recv = pltpu.make_async_remote_copy(
                    src_ref=comm_ref.at[t - 1],
                    dst_ref=comm_ref.at[t - 1],
                    send_sem=send_sems.at[t - 1],
                    recv_sem=recv_sems.at[t - 1],
                    device_id=(my,),
                    device_id_type=pl.DeviceIdType.MESH,
                )
                recv.wait_recv()
                a = comm_ref[t - 1]
            contrib = jnp.dot(
                a, wbuf_ref[t % NBUF].astype(jnp.bfloat16),
                preferred_element_type=jnp.float32,
            )
            if t == 0:
                out_ref[...] = contrib
            else:
                out_ref[...] += contrib
            nxt = t + NBUF
            if nxt < P:
                wcopies[nxt] = w_copy(nxt)
                wcopies[nxt].start()

        for rdma in sends:
            rdma.wait_send()

        y = out_ref[...]
        c = 0.7978845608028654
        out_ref[...] = 0.5 * y * (1.0 + jnp.tanh(c * (y + 0.044715 * y * y * y)))

    return pl.pallas_call(
        body,
        out_shape=jax.ShapeDtypeStruct((m_glob // P, n), jnp.float32),
        in_specs=[
            pl.BlockSpec(memory_space=pltpu.HBM),
            pl.BlockSpec(memory_space=pltpu.HBM),
        ],
        out_specs=pl.BlockSpec(memory_space=pltpu.VMEM),
        scratch_shapes=[
            pltpu.VMEM((m_glob, k_shard), jnp.float32),
            pltpu.VMEM((m_glob, k_shard), jnp.bfloat16),
            pltpu.VMEM((NBUF, K_BLK, n), jnp.float32),
            pltpu.VMEM((P - 1, M_BLK, k_shard), jnp.bfloat16),
            pltpu.SemaphoreType.DMA,
            pltpu.SemaphoreType.DMA((NBUF,)),
            pltpu.SemaphoreType.DMA((P - 1,)),
            pltpu.SemaphoreType.DMA((P - 1,)),
        ],
        compiler_params=pltpu.CompilerParams(
            collective_id=0,
            vmem_limit_bytes=100 * 1024 * 1024,
        ),
    )(x, w_mat)


# device time: 32412 ns/iter; 1.8657x vs baseline; 1.5429x over previous
import jax
import jax.numpy as jnp
from jax import lax
from jax.experimental import pallas as pl
from jax.experimental.pallas import tpu as pltpu

P = 8
M_BLK = 512
K_BLK = 512
NBUF = 3


def kernel(x, w_mat):
    m_glob, k_shard = x.shape
    _k_glob, n = w_mat.shape

    def body(x_hbm, w_hbm, out_ref,
             xf_ref, xq_ref, qscale_ref, comm_ref, rscale_ref, wbuf_ref,
             xsems, wsems, qsend_sems, ssend_sems, qrecv_sems, srecv_sems):
        my = lax.axis_index("i")

        xcopies = {}
        for t in list(range(1, P)) + [0]:
            d = lax.rem(my + t, P)
            xcopies[t] = pltpu.make_async_copy(
                x_hbm.at[pl.ds(d * M_BLK, M_BLK), :],
                xf_ref.at[t],
                xsems.at[t],
            )
            xcopies[t].start()

        def w_copy(idx):
            s = lax.rem(my - idx + P, P)
            return pltpu.make_async_copy(
                w_hbm.at[pl.ds(s * K_BLK, K_BLK), :],
                wbuf_ref.at[idx % NBUF],
                wsems.at[idx % NBUF],
            )

        wcopies = {}
        for idx in range(NBUF):
            wcopies[idx] = w_copy(idx)
            wcopies[idx].start()

        barrier = pltpu.get_barrier_semaphore()
        for t in range(1, P):
            pl.semaphore_signal(
                barrier, inc=1,
                device_id=(lax.rem(my + t, P),),
                device_id_type=pl.DeviceIdType.MESH,
            )
        pl.semaphore_wait(barrier, P - 1)

        sends = []
        for t in range(1, P):
            dst = lax.rem(my + t, P)
            xcopies[t].wait()
            blk = xf_ref[t]
            absmax = jnp.max(jnp.abs(blk), axis=(0, 1), keepdims=True)
            absmax = jnp.maximum(absmax, 1e-20)
            q = jnp.clip(jnp.floor(blk * (127.0 / absmax) + 0.5), -127, 127)
            xq_ref[t - 1] = q.astype(jnp.int8)
            qscale_ref[t - 1] = jnp.broadcast_to(absmax * (1.0 / 127.0), (8, 128))
            qrdma = pltpu.make_async_remote_copy(
                src_ref=xq_ref.at[t - 1],
                dst_ref=comm_ref.at[t - 1],
                send_sem=qsend_sems.at[t - 1],
                recv_sem=qrecv_sems.at[t - 1],
                device_id=(dst,),
                device_id_type=pl.DeviceIdType.MESH,
            )
            qrdma.start()
            srdma = pltpu.make_async_remote_copy(
                src_ref=qscale_ref.at[t - 1],
                dst_ref=rscale_ref.at[t - 1],
                send_sem=ssend_sems.at[t - 1],
                recv_sem=srecv_sems.at[t - 1],
                device_id=(dst,),
                device_id_type=pl.DeviceIdType.MESH,
            )
            srdma.start()
            sends += [qrdma, srdma]

        xcopies[0].wait()
        wcopies[0].wait()
        own = xf_ref[0].astype(jnp.bfloat16)
        out_ref[...] = jnp.dot(
            own, wbuf_ref[0].astype(jnp.bfloat16),
            preferred_element_type=jnp.float32,
        )
        if NBUF < P:
            wcopies[NBUF] = w_copy(NBUF)
            wcopies[NBUF].start()

        for t in range(1, P):
            wcopies[t].wait()
            qrecv = pltpu.make_async_remote_copy(
                src_ref=comm_ref.at[t - 1],
                dst_ref=comm_ref.at[t - 1],
                send_sem=qsend_sems.at[t - 1],
                recv_sem=qrecv_sems.at[t - 1],
                device_id=(my,),
                device_id_type=pl.DeviceIdType.MESH,
            )
            qrecv.wait_recv()
            srecv = pltpu.make_async_remote_copy(
                src_ref=rscale_ref.at[t - 1],
                dst_ref=rscale_ref.at[t - 1],
                send_sem=ssend_sems.at[t - 1],
                recv_sem=srecv_sems.at[t - 1],
                device_id=(my,),
                device_id_type=pl.DeviceIdType.MESH,
            )
            srecv.wait_recv()
            qb = comm_ref[t - 1].astype(jnp.bfloat16)
            contrib = jnp.dot(
                qb, wbuf_ref[t % NBUF].astype(jnp.bfloat16),
                preferred_element_type=jnp.float32,
            )
            out_ref[...] += contrib * rscale_ref[t - 1, 0:1, 0:1]
            nxt = t + NBUF
            if nxt < P:
                wcopies[nxt] = w_copy(nxt)
                wcopies[nxt].start()

        for rdma in sends:
            rdma.wait_send()

        y = out_ref[...]
        c = 0.7978845608028654
        out_ref[...] = 0.5 * y * (1.0 + jnp.tanh(c * (y + 0.044715 * y * y * y)))

    return pl.pallas_call(
        body,
        out_shape=jax.ShapeDtypeStruct((m_glob // P, n), jnp.float32),
        in_specs=[
            pl.BlockSpec(memory_space=pltpu.HBM),
            pl.BlockSpec(memory_space=pltpu.HBM),
        ],
        out_specs=pl.BlockSpec(memory_space=pltpu.VMEM),
        scratch_shapes=[
            pltpu.VMEM((P, M_BLK, k_shard), jnp.float32),
            pltpu.VMEM((P - 1, M_BLK, k_shard), jnp.int8),
            pltpu.VMEM((P - 1, 8, 128), jnp.float32),
            pltpu.VMEM((P - 1, M_BLK, k_shard), jnp.int8),
            pltpu.VMEM((P - 1, 8, 128), jnp.float32),
            pltpu.VMEM((NBUF, K_BLK, n), jnp.float32),
            pltpu.SemaphoreType.DMA((P,)),
            pltpu.SemaphoreType.DMA((NBUF,)),
            pltpu.SemaphoreType.DMA((P - 1,)),
            pltpu.SemaphoreType.DMA((P - 1,)),
            pltpu.SemaphoreType.DMA((P - 1,)),
            pltpu.SemaphoreType.DMA((P - 1,)),
        ],
        compiler_params=pltpu.CompilerParams(
            collective_id=0,
            vmem_limit_bytes=100 * 1024 * 1024,
        ),
    )(x, w_mat)


# device time: 31876 ns/iter; 1.8971x vs baseline; 1.0168x over previous
import jax
import jax.numpy as jnp
from jax import lax
from jax.experimental import pallas as pl
from jax.experimental.pallas import tpu as pltpu

P = 8
M_BLK = 512
K_BLK = 512
NBUF = 3


def _gelu(y):
    c = 0.7978845608028654
    return 0.5 * y * (1.0 + jnp.tanh(c * (y + 0.044715 * y * y * y)))


def kernel(x, w_mat):
    m_glob, k_shard = x.shape
    _k_glob, n = w_mat.shape
    n_half = n // 2

    def body(x_hbm, w_hbm, out_hbm,
             xf_ref, xq_ref, qscale_ref, comm_ref, rscale_ref, wbuf_ref,
             acc_ref, y_ref,
             xsems, wsems, qsend_sems, ssend_sems, qrecv_sems, srecv_sems,
             ysems):
        my = lax.axis_index("i")

        xcopies = {}
        for t in list(range(1, P)) + [0]:
            d = lax.rem(my + t, P)
            xcopies[t] = pltpu.make_async_copy(
                x_hbm.at[pl.ds(d * M_BLK, M_BLK), :],
                xf_ref.at[t],
                xsems.at[t],
            )
            xcopies[t].start()

        def w_copy(idx):
            s = lax.rem(my - idx + P, P)
            return pltpu.make_async_copy(
                w_hbm.at[pl.ds(s * K_BLK, K_BLK), :],
                wbuf_ref.at[idx % NBUF],
                wsems.at[idx % NBUF],
            )

        wcopies = {}
        for idx in range(NBUF):
            wcopies[idx] = w_copy(idx)
            wcopies[idx].start()

        barrier = pltpu.get_barrier_semaphore()
        for t in range(1, P):
            pl.semaphore_signal(
                barrier, inc=1,
                device_id=(lax.rem(my + t, P),),
                device_id_type=pl.DeviceIdType.MESH,
            )
        pl.semaphore_wait(barrier, P - 1)

        sends = []
        for t in range(1, P):
            dst = lax.rem(my + t, P)
            xcopies[t].wait()
            blk = xf_ref[t]
            absmax = jnp.max(jnp.abs(blk), axis=(0, 1), keepdims=True)
            absmax = jnp.maximum(absmax, 1e-20)
            q = jnp.clip(jnp.floor(blk * (127.0 / absmax) + 0.5), -127, 127)
            xq_ref[t - 1] = q.astype(jnp.int8)
            qscale_ref[t - 1] = jnp.broadcast_to(absmax * (1.0 / 127.0), (8, 128))
            qrdma = pltpu.make_async_remote_copy(
                src_ref=xq_ref.at[t - 1],
                dst_ref=comm_ref.at[t - 1],
                send_sem=qsend_sems.at[t - 1],
                recv_sem=qrecv_sems.at[t - 1],
                device_id=(dst,),
                device_id_type=pl.DeviceIdType.MESH,
            )
            qrdma.start()
            srdma = pltpu.make_async_remote_copy(
                src_ref=qscale_ref.at[t - 1],
                dst_ref=rscale_ref.at[t - 1],
                send_sem=ssend_sems.at[t - 1],
                recv_sem=srecv_sems.at[t - 1],
                device_id=(dst,),
                device_id_type=pl.DeviceIdType.MESH,
            )
            srdma.start()
            sends += [qrdma, srdma]

        xcopies[0].wait()
        wcopies[0].wait()
        own = xf_ref[0].astype(jnp.bfloat16)
        acc_ref[...] = jnp.dot(
            own, wbuf_ref[0].astype(jnp.bfloat16),
            preferred_element_type=jnp.float32,
        )
        wcopies[NBUF] = w_copy(NBUF)
        wcopies[NBUF].start()

        def recv_block(t):
            qrecv = pltpu.make_async_remote_copy(
                src_ref=comm_ref.at[t - 1],
                dst_ref=comm_ref.at[t - 1],
                send_sem=qsend_sems.at[t - 1],
                recv_sem=qrecv_sems.at[t - 1],
                device_id=(my,),
                device_id_type=pl.DeviceIdType.MESH,
            )
            qrecv.wait_recv()
            srecv = pltpu.make_async_remote_copy(
                src_ref=rscale_ref.at[t - 1],
                dst_ref=rscale_ref.at[t - 1],
                send_sem=ssend_sems.at[t - 1],
                recv_sem=srecv_sems.at[t - 1],
                device_id=(my,),
                device_id_type=pl.DeviceIdType.MESH,
            )
            srecv.wait_recv()

        for t in range(1, P - 1):
            wcopies[t].wait()
            recv_block(t)
            qb = comm_ref[t - 1].astype(jnp.bfloat16)
            contrib = jnp.dot(
                qb, wbuf_ref[t % NBUF].astype(jnp.bfloat16),
                preferred_element_type=jnp.float32,
            )
            acc_ref[...] += contrib * rscale_ref[t - 1, 0:1, 0:1]
            nxt = t + NBUF
            if nxt < P:
                wcopies[nxt] = w_copy(nxt)
                wcopies[nxt].start()

        t = P - 1
        wcopies[t].wait()
        recv_block(t)
        qb = comm_ref[t - 1].astype(jnp.bfloat16)
        sc = rscale_ref[t - 1, 0:1, 0:1]
        ydmas = []
        for h in range(2):
            cols = slice(h * n_half, (h + 1) * n_half)
            contrib = jnp.dot(
                qb, wbuf_ref[t % NBUF][:, cols].astype(jnp.bfloat16),
                preferred_element_type=jnp.float32,
            )
            y_ref[h] = _gelu(acc_ref[:, cols] + contrib * sc)
            ydma = pltpu.make_async_copy(
                y_ref.at[h],
                out_hbm.at[:, pl.ds(h * n_half, n_half)],
                ysems.at[h],
            )
            ydma.start()
            ydmas.append(ydma)

        for rdma in sends:
            rdma.wait_send()
        for ydma in ydmas:
            ydma.wait()

    return pl.pallas_call(
        body,
        out_shape=jax.ShapeDtypeStruct((m_glob // P, n), jnp.float32),
        in_specs=[
            pl.BlockSpec(memory_space=pltpu.HBM),
            pl.BlockSpec(memory_space=pltpu.HBM),
        ],
        out_specs=pl.BlockSpec(memory_space=pltpu.HBM),
        scratch_shapes=[
            pltpu.VMEM((P, M_BLK, k_shard), jnp.float32),
            pltpu.VMEM((P - 1, M_BLK, k_shard), jnp.int8),
            pltpu.VMEM((P - 1, 8, 128), jnp.float32),
            pltpu.VMEM((P - 1, M_BLK, k_shard), jnp.int8),
            pltpu.VMEM((P - 1, 8, 128), jnp.float32),
            pltpu.VMEM((NBUF, K_BLK, n), jnp.float32),
            pltpu.VMEM((M_BLK, n), jnp.float32),
            pltpu.VMEM((2, M_BLK, n // 2), jnp.float32),
            pltpu.SemaphoreType.DMA((P,)),
            pltpu.SemaphoreType.DMA((NBUF,)),
            pltpu.SemaphoreType.DMA((P - 1,)),
            pltpu.SemaphoreType.DMA((P - 1,)),
            pltpu.SemaphoreType.DMA((P - 1,)),
            pltpu.SemaphoreType.DMA((P - 1,)),
            pltpu.SemaphoreType.DMA((2,)),
        ],
        compiler_params=pltpu.CompilerParams(
            collective_id=0,
            vmem_limit_bytes=100 * 1024 * 1024,
        ),
    )(x, w_mat)


# device time: 30128 ns/iter; 2.0071x vs baseline; 1.0580x over previous
import jax
import jax.numpy as jnp
from jax import lax
from jax.experimental import pallas as pl
from jax.experimental.pallas import tpu as pltpu

P = 8
M_BLK = 512
K_BLK = 512
NBUF = 3


def _gelu(y):
    c = 0.7978845608028654
    return 0.5 * y * (1.0 + jnp.tanh(c * (y + 0.044715 * y * y * y)))


def kernel(x, w_mat):
    m_glob, k_shard = x.shape
    _k_glob, n = w_mat.shape
    n_half = n // 2

    def body(x_hbm, w_hbm, out_hbm,
             xf_ref, xq_ref, qscale_ref, comm_ref, rscale_ref, wbuf_ref,
             acc_ref, y_ref,
             xsems, wsems, qsend_sems, ssend_sems, qrecv_sems, srecv_sems,
             ysems):
        my = lax.axis_index("i")

        xcopies = {}
        for t in list(range(1, P)) + [0]:
            d = lax.rem(my + t, P)
            xcopies[t] = pltpu.make_async_copy(
                x_hbm.at[pl.ds(d * M_BLK, M_BLK), :],
                xf_ref.at[t],
                xsems.at[t],
            )
            xcopies[t].start()

        def w_copy(idx):
            s = lax.rem(my - idx + P, P)
            return pltpu.make_async_copy(
                w_hbm.at[pl.ds(s * K_BLK, K_BLK), :],
                wbuf_ref.at[idx % NBUF],
                wsems.at[idx % NBUF],
            )

        wcopies = {}
        for idx in range(NBUF):
            wcopies[idx] = w_copy(idx)
            wcopies[idx].start()

        barrier = pltpu.get_barrier_semaphore()
        for t in range(1, P):
            pl.semaphore_signal(
                barrier, inc=1,
                device_id=(lax.rem(my + t, P),),
                device_id_type=pl.DeviceIdType.MESH,
            )
        pl.semaphore_wait(barrier, P - 1)

        sends = []
        for t in range(1, P):
            dst = lax.rem(my + t, P)
            xcopies[t].wait()
            blk = xf_ref[t]
            absmax = jnp.max(jnp.abs(blk), axis=(0, 1), keepdims=True)
            absmax = jnp.maximum(absmax, 1e-20)
            q = jnp.clip(jnp.floor(blk * (127.0 / absmax) + 0.5), -127, 127)
            xq_ref[t - 1] = q.astype(jnp.int8)
            qscale_ref[t - 1] = jnp.broadcast_to(absmax * (1.0 / 127.0), (8, 128))
            qrdma = pltpu.make_async_remote_copy(
                src_ref=xq_ref.at[t - 1],
                dst_ref=comm_ref.at[t - 1],
                send_sem=qsend_sems.at[t - 1],
                recv_sem=qrecv_sems.at[t - 1],
                device_id=(dst,),
                device_id_type=pl.DeviceIdType.MESH,
            )
            qrdma.start()
            srdma = pltpu.make_async_remote_copy(
                src_ref=qscale_ref.at[t - 1],
                dst_ref=rscale_ref.at[t - 1],
                send_sem=ssend_sems.at[t - 1],
                recv_sem=srecv_sems.at[t - 1],
                device_id=(dst,),
                device_id_type=pl.DeviceIdType.MESH,
            )
            srdma.start()
            sends += [qrdma, srdma]

        xcopies[0].wait()
        wcopies[0].wait()
        own = xf_ref[0].astype(jnp.bfloat16)
        acc_ref[...] = jnp.dot(
            own, wbuf_ref[0].astype(jnp.bfloat16),
            preferred_element_type=jnp.float32,
        )
        wcopies[NBUF] = w_copy(NBUF)
        wcopies[NBUF].start()

        def recv_block(t):
            qrecv = pltpu.make_async_remote_copy(
                src_ref=comm_ref.at[t - 1],
                dst_ref=comm_ref.at[t - 1],
                send_sem=qsend_sems.at[t - 1],
                recv_sem=qrecv_sems.at[t - 1],
                device_id=(my,),
                device_id_type=pl.DeviceIdType.MESH,
            )
            qrecv.wait_recv()
            srecv = pltpu.make_async_remote_copy(
                src_ref=rscale_ref.at[t - 1],
                dst_ref=rscale_ref.at[t - 1],
                send_sem=ssend_sems.at[t - 1],
                recv_sem=srecv_sems.at[t - 1],
                device_id=(my,),
                device_id_type=pl.DeviceIdType.MESH,
            )
            srecv.wait_recv()

        for t in range(1, P - 1):
            wcopies[t].wait()
            recv_block(t)
            qb = comm_ref[t - 1].astype(jnp.bfloat16)
            contrib = jnp.dot(
                qb, wbuf_ref[t % NBUF].astype(jnp.bfloat16),
                preferred_element_type=jnp.float32,
            )
            acc_ref[...] += contrib * rscale_ref[t - 1, 0:1, 0:1]
            nxt = t + NBUF
            if nxt < P:
                wcopies[nxt] = w_copy(nxt)
                wcopies[nxt].start()

        t = P - 1
        wcopies[t].wait()
        recv_block(t)
        qb = comm_ref[t - 1].astype(jnp.bfloat16)
        sc = rscale_ref[t - 1, 0:1, 0:1]
        ydmas = []
        for h in range(2):
            cols = slice(h * n_half, (h + 1) * n_half)
            contrib = jnp.dot(
                qb, wbuf_ref[t % NBUF][:, cols].astype(jnp.bfloat16),
                preferred_element_type=jnp.float32,
            )
            y_ref[h] = _gelu(acc_ref[:, cols] + contrib * sc).astype(jnp.bfloat16)
            ydma = pltpu.make_async_copy(
                y_ref.at[h],
                out_hbm.at[:, pl.ds(h * n_half, n_half)],
                ysems.at[h],
            )
            ydma.start()
            ydmas.append(ydma)

        for rdma in sends:
            rdma.wait_send()
        for ydma in ydmas:
            ydma.wait()

    return pl.pallas_call(
        body,
        out_shape=jax.ShapeDtypeStruct((m_glob // P, n), jnp.bfloat16),
        in_specs=[
            pl.BlockSpec(memory_space=pltpu.HBM),
            pl.BlockSpec(memory_space=pltpu.HBM),
        ],
        out_specs=pl.BlockSpec(memory_space=pltpu.HBM),
        scratch_shapes=[
            pltpu.VMEM((P, M_BLK, k_shard), jnp.float32),
            pltpu.VMEM((P - 1, M_BLK, k_shard), jnp.int8),
            pltpu.VMEM((P - 1, 8, 128), jnp.float32),
            pltpu.VMEM((P - 1, M_BLK, k_shard), jnp.int8),
            pltpu.VMEM((P - 1, 8, 128), jnp.float32),
            pltpu.VMEM((NBUF, K_BLK, n), jnp.float32),
            pltpu.VMEM((M_BLK, n), jnp.float32),
            pltpu.VMEM((2, M_BLK, n // 2), jnp.bfloat16),
            pltpu.SemaphoreType.DMA((P,)),
            pltpu.SemaphoreType.DMA((NBUF,)),
            pltpu.SemaphoreType.DMA((P - 1,)),
            pltpu.SemaphoreType.DMA((P - 1,)),
            pltpu.SemaphoreType.DMA((P - 1,)),
            pltpu.SemaphoreType.DMA((P - 1,)),
            pltpu.SemaphoreType.DMA((2,)),
        ],
        compiler_params=pltpu.CompilerParams(
            collective_id=0,
            vmem_limit_bytes=100 * 1024 * 1024,
        ),
    )(x, w_mat)


# device time: 29938 ns/iter; 2.0199x vs baseline; 1.0063x over previous
import jax
import jax.numpy as jnp
from jax import lax
from jax.experimental import pallas as pl
from jax.experimental.pallas import tpu as pltpu

P = 8
M_BLK = 512
K_BLK = 512
NBUF = 3
N_TAIL = 4


def _gelu(y):
    c = 0.7978845608028654
    return 0.5 * y * (1.0 + jnp.tanh(c * (y + 0.044715 * y * y * y)))


def kernel(x, w_mat):
    m_glob, k_shard = x.shape
    _k_glob, n = w_mat.shape

    def body(x_hbm, w_hbm, out_hbm,
             xf_ref, xq_ref, qscale_ref, comm_ref, rscale_ref, wbuf_ref,
             acc_ref, y_ref,
             xsems, wsems, qsend_sems, ssend_sems, qrecv_sems, srecv_sems,
             ysems):
        my = lax.axis_index("i")

        xcopies = {}
        for t in list(range(1, P)) + [0]:
            d = lax.rem(my + t, P)
            xcopies[t] = pltpu.make_async_copy(
                x_hbm.at[pl.ds(d * M_BLK, M_BLK), :],
                xf_ref.at[t],
                xsems.at[t],
            )
            xcopies[t].start()

        def w_copy(idx):
            s = lax.rem(my - idx + P, P)
            return pltpu.make_async_copy(
                w_hbm.at[pl.ds(s * K_BLK, K_BLK), :],
                wbuf_ref.at[idx % NBUF],
                wsems.at[idx % NBUF],
            )

        wcopies = {}
        for idx in range(NBUF):
            wcopies[idx] = w_copy(idx)
            wcopies[idx].start()

        barrier = pltpu.get_barrier_semaphore()
        for t in range(1, P):
            pl.semaphore_signal(
                barrier, inc=1,
                device_id=(lax.rem(my + t, P),),
                device_id_type=pl.DeviceIdType.MESH,
            )
        pl.semaphore_wait(barrier, P - 1)

        sends = []
        for t in range(1, P):
            dst = lax.rem(my + t, P)
            xcopies[t].wait()
            blk = xf_ref[t]
            absmax = jnp.max(jnp.abs(blk), axis=(0, 1), keepdims=True)
            absmax = jnp.maximum(absmax, 1e-20)
            q = jnp.clip(jnp.floor(blk * (127.0 / absmax) + 0.5), -127, 127)
            xq_ref[t - 1] = q.astype(jnp.int8)
            qscale_ref[t - 1] = jnp.broadcast_to(absmax * (1.0 / 127.0), (8, 128))
            qrdma = pltpu.make_async_remote_copy(
                src_ref=xq_ref.at[t - 1],
                dst_ref=comm_ref.at[t - 1],
                send_sem=qsend_sems.at[t - 1],
                recv_sem=qrecv_sems.at[t - 1],
                device_id=(dst,),
                device_id_type=pl.DeviceIdType.MESH,
            )
            qrdma.start()
            srdma = pltpu.make_async_remote_copy(
                src_ref=qscale_ref.at[t - 1],
                dst_ref=rscale_ref.at[t - 1],
                send_sem=ssend_sems.at[t - 1],
                recv_sem=srecv_sems.at[t - 1],
                device_id=(dst,),
                device_id_type=pl.DeviceIdType.MESH,
            )
            srdma.start()
            sends += [qrdma, srdma]

        xcopies[0].wait()
        wcopies[0].wait()
        own = xf_ref[0].astype(jnp.bfloat16)
        acc_ref[...] = jnp.dot(
            own, wbuf_ref[0].astype(jnp.bfloat16),
            preferred_element_type=jnp.float32,
        )
        wcopies[NBUF] = w_copy(NBUF)
        wcopies[NBUF].start()

        def recv_block(t):
            qrecv = pltpu.make_async_remote_copy(
                src_ref=comm_ref.at[t - 1],
                dst_ref=comm_ref.at[t - 1],
                send_sem=qsend_sems.at[t - 1],
                recv_sem=qrecv_sems.at[t - 1],
                device_id=(my,),
                device_id_type=pl.DeviceIdType.MESH,
            )
            qrecv.wait_recv()
            srecv = pltpu.make_async_remote_copy(
                src_ref=rscale_ref.at[t - 1],
                dst_ref=rscale_ref.at[t - 1],
                send_sem=ssend_sems.at[t - 1],
                recv_sem=srecv_sems.at[t - 1],
                device_id=(my,),
                device_id_type=pl.DeviceIdType.MESH,
            )
            srecv.wait_recv()

        for t in range(1, P - 1):
            wcopies[t].wait()
            recv_block(t)
            qb = comm_ref[t - 1].astype(jnp.bfloat16)
            contrib = jnp.dot(
                qb, wbuf_ref[t % NBUF].astype(jnp.bfloat16),
                preferred_element_type=jnp.float32,
            )
            acc_ref[...] += contrib * rscale_ref[t - 1, 0:1, 0:1]
            nxt = t + NBUF
            if nxt < P:
                wcopies[nxt] = w_copy(nxt)
                wcopies[nxt].start()

        t = P - 1
        wcopies[t].wait()
        recv_block(t)
        qb = comm_ref[t - 1].astype(jnp.bfloat16)
        sc = rscale_ref[t - 1, 0:1, 0:1]
        ydmas = []
        nch = n // N_TAIL
        for h in range(N_TAIL):
            cols = slice(h * nch, (h + 1) * nch)
            contrib = jnp.dot(
                qb, wbuf_ref[t % NBUF][:, cols].astype(jnp.bfloat16),
                preferred_element_type=jnp.float32,
            )
            y_ref[h] = _gelu(acc_ref[:, cols] + contrib * sc).astype(jnp.bfloat16)
            ydma = pltpu.make_async_copy(
                y_ref.at[h],
                out_hbm.at[:, pl.ds(h * nch, nch)],
                ysems.at[h],
            )
            ydma.start()
            ydmas.append(ydma)

        for rdma in sends:
            rdma.wait_send()
        for ydma in ydmas:
            ydma.wait()

    return pl.pallas_call(
        body,
        out_shape=jax.ShapeDtypeStruct((m_glob // P, n), jnp.bfloat16),
        in_specs=[
            pl.BlockSpec(memory_space=pltpu.HBM),
            pl.BlockSpec(memory_space=pltpu.HBM),
        ],
        out_specs=pl.BlockSpec(memory_space=pltpu.HBM),
        scratch_shapes=[
            pltpu.VMEM((P, M_BLK, k_shard), jnp.float32),
            pltpu.VMEM((P - 1, M_BLK, k_shard), jnp.int8),
            pltpu.VMEM((P - 1, 8, 128), jnp.float32),
            pltpu.VMEM((P - 1, M_BLK, k_shard), jnp.int8),
            pltpu.VMEM((P - 1, 8, 128), jnp.float32),
            pltpu.VMEM((NBUF, K_BLK, n), jnp.float32),
            pltpu.VMEM((M_BLK, n), jnp.float32),
            pltpu.VMEM((N_TAIL, M_BLK, n // N_TAIL), jnp.bfloat16),
            pltpu.SemaphoreType.DMA((P,)),
            pltpu.SemaphoreType.DMA((NBUF,)),
            pltpu.SemaphoreType.DMA((P - 1,)),
            pltpu.SemaphoreType.DMA((P - 1,)),
            pltpu.SemaphoreType.DMA((P - 1,)),
            pltpu.SemaphoreType.DMA((P - 1,)),
            pltpu.SemaphoreType.DMA((N_TAIL,)),
        ],
        compiler_params=pltpu.CompilerParams(
            collective_id=0,
            vmem_limit_bytes=100 * 1024 * 1024,
        ),
    )(x, w_mat)
